# Initial kernel scaffold; baseline (speedup 1.0000x reference)
import functools

import jax
import jax.numpy as jnp
from jax import lax
from jax.experimental import pallas as pl
from jax.experimental.pallas import tpu as pltpu
from jax.experimental.pallas import tpu_sc as plsc

N = 10000
E = 320000
D_IN = 128
D_LAT = 64

NC = 2
NS = 16
NW = NC * NS
CHUNK = 128
EPT = E // NW
EPT_PAD = ((EPT + CHUNK - 1) // CHUNK) * CHUNK
E_PAD = EPT_PAD * NW
NCHUNKS = EPT_PAD // CHUNK
ROWS_PER_TILE = N // NS

_mesh = plsc.VectorSubcoreMesh(
    core_axis_name="c", subcore_axis_name="s", num_cores=NC, num_subcores=NS)


@functools.partial(
    pl.kernel,
    out_type=[jax.ShapeDtypeStruct((N, D_IN), jnp.float32),
              jax.ShapeDtypeStruct((N, D_IN), jnp.float32)],
    mesh=_mesh,
    scratch_types=[
        pltpu.VMEM((CHUNK,), jnp.int32),
        pltpu.VMEM((CHUNK,), jnp.int32),
        pltpu.VMEM((CHUNK,), jnp.float32),
        pltpu.VMEM((CHUNK, D_IN), jnp.float32),
        pltpu.VMEM_SHARED((N, D_IN), jnp.float32),
        pltpu.SemaphoreType.DMA,
    ],
)
def _spmm_kernel(x_hbm, rows_hbm, cols_hbm, vals_hbm, zeros_hbm,
                 out0_hbm, out1_hbm,
                 rows_v, cols_v, vals_v, xb, agg_sh, sem):
    cid = lax.axis_index("c")
    sid = lax.axis_index("s")
    wid = cid * NS + sid

    pltpu.sync_copy(zeros_hbm, agg_sh.at[pl.ds(sid * ROWS_PER_TILE,
                                               ROWS_PER_TILE)])
    plsc.subcore_barrier()

    def chunk_body(k, carry):
        base = wid * EPT_PAD + k * CHUNK
        pltpu.sync_copy(rows_hbm.at[pl.ds(base, CHUNK)], rows_v)
        pltpu.sync_copy(cols_hbm.at[pl.ds(base, CHUNK)], cols_v)
        pltpu.sync_copy(vals_hbm.at[pl.ds(base, CHUNK)], vals_v)
        pltpu.async_copy(x_hbm.at[cols_v], xb, sem).wait()

        def edge_body(i, c2):
            v = vals_v[i]
            for j in range(D_IN // 16):
                xb[i, pl.ds(j * 16, 16)] = xb[i, pl.ds(j * 16, 16)] * v
            return c2
        lax.fori_loop(0, CHUNK, edge_body, 0)

        pltpu.sync_copy(xb, agg_sh.at[rows_v], add=True)
        return carry
    lax.fori_loop(0, NCHUNKS, chunk_body, 0)

    plsc.subcore_barrier()
    sl = pl.ds(sid * ROWS_PER_TILE, ROWS_PER_TILE)

    @pl.when(cid == 0)
    def _():
        pltpu.sync_copy(agg_sh.at[sl], out0_hbm.at[sl])

    @pl.when(cid == 1)
    def _():
        pltpu.sync_copy(agg_sh.at[sl], out1_hbm.at[sl])


def _heads_body(p0_ref, p1_ref, wmu_ref, bmu_ref, wlv_ref, blv_ref,
                mu_ref, lv_ref):
    agg = p0_ref[...] + p1_ref[...]
    dn = (((1,), (1,)), ((), ()))
    mu = lax.dot_general(agg, wmu_ref[...], dn,
                         preferred_element_type=jnp.float32) + bmu_ref[...]
    lv = lax.dot_general(agg, wlv_ref[...], dn,
                         preferred_element_type=jnp.float32) + blv_ref[...]
    mu_ref[...] = mu
    lv_ref[...] = jnp.clip(lv, -10.0, 3.0)


_heads_call = pl.pallas_call(
    _heads_body,
    out_shape=[jax.ShapeDtypeStruct((N, D_LAT), jnp.float32),
               jax.ShapeDtypeStruct((N, D_LAT), jnp.float32)],
)


@functools.partial(
    pl.kernel,
    out_type=jax.ShapeDtypeStruct((E_PAD,), jnp.float32),
    mesh=_mesh,
    scratch_types=[
        pltpu.VMEM((CHUNK,), jnp.int32),
        pltpu.VMEM((CHUNK,), jnp.int32),
        pltpu.VMEM((CHUNK, D_LAT), jnp.float32),
        pltpu.VMEM((CHUNK, D_LAT), jnp.float32),
        pltpu.VMEM((CHUNK,), jnp.float32),
        pltpu.SemaphoreType.DMA,
        pltpu.SemaphoreType.DMA,
    ],
)
def _decoder_kernel(z_hbm, r_hbm, c_hbm, dec_hbm,
                    r_v, c_v, zr, zc, dec_v, sem_r, sem_c):
    cid = lax.axis_index("c")
    sid = lax.axis_index("s")
    wid = cid * NS + sid

    def chunk_body(k, carry):
        base = wid * EPT_PAD + k * CHUNK
        pltpu.sync_copy(r_hbm.at[pl.ds(base, CHUNK)], r_v)
        pltpu.sync_copy(c_hbm.at[pl.ds(base, CHUNK)], c_v)
        cp_r = pltpu.async_copy(z_hbm.at[r_v], zr, sem_r)
        cp_c = pltpu.async_copy(z_hbm.at[c_v], zc, sem_c)
        cp_r.wait()
        cp_c.wait()

        def group_body(g, c2):
            rows16 = lax.iota(jnp.int32, (16,)) + g * 16
            acc = jnp.zeros((16,), jnp.float32)
            for j in range(D_LAT):
                colj = jnp.full((16,), j, jnp.int32)
                a = plsc.load_gather(zr, [rows16, colj])
                b = plsc.load_gather(zc, [rows16, colj])
                acc = acc + a * b
            dec_v[pl.ds(g * 16, 16)] = acc
            return c2
        lax.fori_loop(0, CHUNK // 16, group_body, 0)

        pltpu.sync_copy(dec_v, dec_hbm.at[pl.ds(base, CHUNK)])
        return carry
    lax.fori_loop(0, NCHUNKS, chunk_body, 0)


def kernel(x, adj_edge_index, adj_values, edge_index, W_mu, b_mu,
           W_logvar, b_logvar):
    pad = E_PAD - E
    ar = jnp.concatenate([adj_edge_index[0].astype(jnp.int32),
                          jnp.zeros((pad,), jnp.int32)])
    ac = jnp.concatenate([adj_edge_index[1].astype(jnp.int32),
                          jnp.zeros((pad,), jnp.int32)])
    av = jnp.concatenate([adj_values.astype(jnp.float32),
                          jnp.zeros((pad,), jnp.float32)])
    er = jnp.concatenate([edge_index[0].astype(jnp.int32),
                          jnp.zeros((pad,), jnp.int32)])
    ec = jnp.concatenate([edge_index[1].astype(jnp.int32),
                          jnp.zeros((pad,), jnp.int32)])
    zeros_blk = jnp.zeros((ROWS_PER_TILE, D_IN), jnp.float32)

    p0, p1 = _spmm_kernel(x, ar, ac, av, zeros_blk)
    mu, logvar = _heads_call(p0, p1, W_mu, b_mu.reshape(1, D_LAT),
                             W_logvar, b_logvar.reshape(1, D_LAT))
    dec_pad = _decoder_kernel(mu, er, ec)
    return (dec_pad[:E], mu, logvar)


# speedup vs baseline: 3.3872x; 3.3872x over previous
import functools

import jax
import jax.numpy as jnp
from jax import lax
from jax.experimental import pallas as pl
from jax.experimental.pallas import tpu as pltpu
from jax.experimental.pallas import tpu_sc as plsc

N = 10000
E = 320000
D_IN = 128
D_LAT = 64

NC = 2
NS = 16
NW = NC * NS
CHUNK = 128
EPT = E // NW
EPT_PAD = ((EPT + CHUNK - 1) // CHUNK) * CHUNK
E_PAD = EPT_PAD * NW
NCHUNKS = EPT_PAD // CHUNK
STRIPE = 640
LAST_STRIPE = N - 15 * STRIPE

_mesh = plsc.VectorSubcoreMesh(
    core_axis_name="c", subcore_axis_name="s", num_cores=NC, num_subcores=NS)


@functools.partial(
    pl.kernel,
    out_type=[jax.ShapeDtypeStruct((N, D_IN), jnp.float32),
              jax.ShapeDtypeStruct((N, D_IN), jnp.float32)],
    mesh=_mesh,
    scratch_types=[
        pltpu.VMEM((CHUNK,), jnp.int32),
        pltpu.VMEM((CHUNK,), jnp.int32),
        pltpu.VMEM((CHUNK,), jnp.float32),
        pltpu.VMEM((CHUNK, D_IN), jnp.float32),
        pltpu.VMEM_SHARED((N, D_IN), jnp.float32),
        pltpu.SemaphoreType.DMA,
    ],
)
def _spmm_kernel(x_hbm, rows_hbm, cols_hbm, vals_hbm, zeros_hbm,
                 out0_hbm, out1_hbm,
                 rows_v, cols_v, vals_v, xb, agg_sh, sem):
    cid = lax.axis_index("c")
    sid = lax.axis_index("s")
    wid = cid * NS + sid

    @pl.when(sid < NS - 1)
    def _():
        pltpu.sync_copy(zeros_hbm, agg_sh.at[pl.ds(sid * STRIPE, STRIPE)])

    @pl.when(sid == NS - 1)
    def _():
        pltpu.sync_copy(zeros_hbm.at[pl.ds(0, LAST_STRIPE)],
                        agg_sh.at[pl.ds(sid * STRIPE, LAST_STRIPE)])
    plsc.subcore_barrier()

    def chunk_body(k, carry):
        base = wid * EPT_PAD + k * CHUNK
        pltpu.sync_copy(rows_hbm.at[pl.ds(base, CHUNK)], rows_v)
        pltpu.sync_copy(cols_hbm.at[pl.ds(base, CHUNK)], cols_v)
        pltpu.sync_copy(vals_hbm.at[pl.ds(base, CHUNK)], vals_v)
        pltpu.async_copy(x_hbm.at[cols_v], xb, sem).wait()

        def group_body(g, c2):
            w = vals_v[pl.ds(g * 16, 16)]
            for l in range(16):
                v = w[l]
                i = g * 16 + l
                for j in range(D_IN // 16):
                    xb[i, pl.ds(j * 16, 16)] = xb[i, pl.ds(j * 16, 16)] * v
            return c2
        lax.fori_loop(0, CHUNK // 16, group_body, 0)

        pltpu.sync_copy(xb, agg_sh.at[rows_v], add=True)
        return carry
    lax.fori_loop(0, NCHUNKS, chunk_body, 0)

    plsc.subcore_barrier()

    @pl.when(jnp.logical_and(cid == 0, sid < NS - 1))
    def _():
        sl = pl.ds(sid * STRIPE, STRIPE)
        pltpu.sync_copy(agg_sh.at[sl], out0_hbm.at[sl])

    @pl.when(jnp.logical_and(cid == 0, sid == NS - 1))
    def _():
        sl = pl.ds(sid * STRIPE, LAST_STRIPE)
        pltpu.sync_copy(agg_sh.at[sl], out0_hbm.at[sl])

    @pl.when(jnp.logical_and(cid == 1, sid < NS - 1))
    def _():
        sl = pl.ds(sid * STRIPE, STRIPE)
        pltpu.sync_copy(agg_sh.at[sl], out1_hbm.at[sl])

    @pl.when(jnp.logical_and(cid == 1, sid == NS - 1))
    def _():
        sl = pl.ds(sid * STRIPE, LAST_STRIPE)
        pltpu.sync_copy(agg_sh.at[sl], out1_hbm.at[sl])


def _heads_body(p0_ref, p1_ref, wmu_ref, bmu_ref, wlv_ref, blv_ref,
                mu_ref, lv_ref):
    agg = p0_ref[...] + p1_ref[...]
    dn = (((1,), (1,)), ((), ()))
    mu = lax.dot_general(agg, wmu_ref[...], dn,
                         precision=lax.Precision.HIGHEST,
                         preferred_element_type=jnp.float32) + bmu_ref[...]
    lv = lax.dot_general(agg, wlv_ref[...], dn,
                         precision=lax.Precision.HIGHEST,
                         preferred_element_type=jnp.float32) + blv_ref[...]
    mu_ref[...] = mu
    lv_ref[...] = jnp.clip(lv, -10.0, 3.0)


_heads_call = pl.pallas_call(
    _heads_body,
    out_shape=[jax.ShapeDtypeStruct((N, D_LAT), jnp.float32),
               jax.ShapeDtypeStruct((N, D_LAT), jnp.float32)],
)


@functools.partial(
    pl.kernel,
    out_type=jax.ShapeDtypeStruct((E_PAD * 16 // 128, 128), jnp.float32),
    mesh=_mesh,
    scratch_types=[
        pltpu.VMEM((CHUNK,), jnp.int32),
        pltpu.VMEM((CHUNK,), jnp.int32),
        pltpu.VMEM((CHUNK, D_LAT), jnp.float32),
        pltpu.VMEM((CHUNK, D_LAT), jnp.float32),
        pltpu.VMEM((CHUNK * 16 // 128, 128), jnp.float32),
        pltpu.SemaphoreType.DMA,
        pltpu.SemaphoreType.DMA,
    ],
    compiler_params=pltpu.CompilerParams(use_tc_tiling_on_sc=False),
)
def _decoder_kernel(z_hbm, r_hbm, c_hbm, ph_hbm,
                    r_v, c_v, zr, zc, sbuf, sem_r, sem_c):
    cid = lax.axis_index("c")
    sid = lax.axis_index("s")
    wid = cid * NS + sid

    def chunk_body(k, carry):
        base = wid * EPT_PAD + k * CHUNK
        pltpu.sync_copy(r_hbm.at[pl.ds(base, CHUNK)], r_v)
        pltpu.sync_copy(c_hbm.at[pl.ds(base, CHUNK)], c_v)
        cp_r = pltpu.async_copy(z_hbm.at[r_v], zr, sem_r)
        cp_c = pltpu.async_copy(z_hbm.at[c_v], zc, sem_c)
        cp_r.wait()
        cp_c.wait()

        def edge_body(i, c2):
            s = jnp.zeros((16,), jnp.float32)
            for j in range(D_LAT // 16):
                s = s + zr[i, pl.ds(j * 16, 16)] * zc[i, pl.ds(j * 16, 16)]
            sbuf[i // 8, pl.ds((i % 8) * 16, 16)] = s
            return c2
        lax.fori_loop(0, CHUNK, edge_body, 0)

        pltpu.sync_copy(sbuf, ph_hbm.at[pl.ds(base // 8, CHUNK * 16 // 128)])
        return carry
    lax.fori_loop(0, NCHUNKS, chunk_body, 0)


_RBLK = 512
_RROWS = E_PAD * 16 // 128


def _reduce_body(ph_ref, out_ref):
    rows = lax.broadcasted_iota(jnp.int32, (128, 128), 0)
    cols = lax.broadcasted_iota(jnp.int32, (128, 128), 1)
    g = jnp.logical_and(rows // 16 == cols, cols < 8).astype(jnp.float32)
    out_ref[...] = lax.dot_general(ph_ref[...], g, (((1,), (0,)), ((), ())),
                                   precision=lax.Precision.HIGHEST,
                                   preferred_element_type=jnp.float32)


_reduce_call = pl.pallas_call(
    _reduce_body,
    grid=(_RROWS // _RBLK,),
    in_specs=[pl.BlockSpec((_RBLK, 128), lambda i: (i, 0))],
    out_specs=pl.BlockSpec((_RBLK, 128), lambda i: (i, 0)),
    out_shape=jax.ShapeDtypeStruct((_RROWS, 128), jnp.float32),
)


def kernel(x, adj_edge_index, adj_values, edge_index, W_mu, b_mu,
           W_logvar, b_logvar):
    pad = E_PAD - E
    ar = jnp.concatenate([adj_edge_index[0].astype(jnp.int32),
                          jnp.zeros((pad,), jnp.int32)])
    ac = jnp.concatenate([adj_edge_index[1].astype(jnp.int32),
                          jnp.zeros((pad,), jnp.int32)])
    av = jnp.concatenate([adj_values.astype(jnp.float32),
                          jnp.zeros((pad,), jnp.float32)])
    er = jnp.concatenate([edge_index[0].astype(jnp.int32),
                          jnp.zeros((pad,), jnp.int32)])
    ec = jnp.concatenate([edge_index[1].astype(jnp.int32),
                          jnp.zeros((pad,), jnp.int32)])
    zeros_blk = jnp.zeros((STRIPE, D_IN), jnp.float32)

    p0, p1 = _spmm_kernel(x, ar, ac, av, zeros_blk)
    mu, logvar = _heads_call(p0, p1, W_mu, b_mu.reshape(1, D_LAT),
                             W_logvar, b_logvar.reshape(1, D_LAT))
    ph = _decoder_kernel(mu, er, ec)
    dec_pad = _reduce_call(ph)[:, :8].reshape(E_PAD)
    return (dec_pad[:E], mu, logvar)


# speedup vs baseline: 4.0392x; 1.1925x over previous
import functools

import jax
import jax.numpy as jnp
from jax import lax
from jax.experimental import pallas as pl
from jax.experimental.pallas import tpu as pltpu
from jax.experimental.pallas import tpu_sc as plsc

N = 10000
E = 320000
D_IN = 128
D_LAT = 64

NC = 2
NS = 16
NW = NC * NS
CHUNK = 128
EPT = E // NW
EPT_PAD = ((EPT + CHUNK - 1) // CHUNK) * CHUNK
E_PAD = EPT_PAD * NW
NCHUNKS = EPT_PAD // CHUNK
STRIPE = 640
LAST_STRIPE = N - 15 * STRIPE

_mesh = plsc.VectorSubcoreMesh(
    core_axis_name="c", subcore_axis_name="s", num_cores=NC, num_subcores=NS)


@functools.partial(
    pl.kernel,
    out_type=[jax.ShapeDtypeStruct((N, D_IN), jnp.float32),
              jax.ShapeDtypeStruct((N, D_IN), jnp.float32)],
    mesh=_mesh,
    scratch_types=[
        pltpu.VMEM((CHUNK,), jnp.int32),
        pltpu.VMEM((CHUNK,), jnp.int32),
        pltpu.VMEM((CHUNK,), jnp.float32),
        pltpu.VMEM((CHUNK, D_IN), jnp.float32),
        pltpu.VMEM((CHUNK,), jnp.int32),
        pltpu.VMEM((CHUNK,), jnp.int32),
        pltpu.VMEM((CHUNK,), jnp.float32),
        pltpu.VMEM((CHUNK, D_IN), jnp.float32),
        pltpu.VMEM_SHARED((N, D_IN), jnp.float32),
        pltpu.SemaphoreType.DMA,
        pltpu.SemaphoreType.DMA,
    ],
)
def _spmm_kernel(x_hbm, rows_hbm, cols_hbm, vals_hbm, zeros_hbm,
                 out0_hbm, out1_hbm,
                 rows_v0, cols_v0, vals_v0, xb0,
                 rows_v1, cols_v1, vals_v1, xb1, agg_sh, sem0, sem1):
    cid = lax.axis_index("c")
    sid = lax.axis_index("s")
    wid = cid * NS + sid

    @pl.when(sid < NS - 1)
    def _():
        pltpu.sync_copy(zeros_hbm, agg_sh.at[pl.ds(sid * STRIPE, STRIPE)])

    @pl.when(sid == NS - 1)
    def _():
        pltpu.sync_copy(zeros_hbm.at[pl.ds(0, LAST_STRIPE)],
                        agg_sh.at[pl.ds(sid * STRIPE, LAST_STRIPE)])
    plsc.subcore_barrier()

    def _scale_and_scatter(xb, vals_v, rows_v):
        def group_body(g, c2):
            w = vals_v[pl.ds(g * 16, 16)]
            for l in range(16):
                v = w[l]
                i = g * 16 + l
                for j in range(D_IN // 16):
                    xb[i, pl.ds(j * 16, 16)] = xb[i, pl.ds(j * 16, 16)] * v
            return c2
        lax.fori_loop(0, CHUNK // 16, group_body, 0)
        pltpu.sync_copy(xb, agg_sh.at[rows_v], add=True)

    def pair_body(p, carry):
        base_a = wid * EPT_PAD + (2 * p) * CHUNK
        base_b = base_a + CHUNK
        pltpu.sync_copy(rows_hbm.at[pl.ds(base_a, CHUNK)], rows_v0)
        pltpu.sync_copy(cols_hbm.at[pl.ds(base_a, CHUNK)], cols_v0)
        pltpu.sync_copy(vals_hbm.at[pl.ds(base_a, CHUNK)], vals_v0)
        cp_a = pltpu.async_copy(x_hbm.at[cols_v0], xb0, sem0)
        pltpu.sync_copy(rows_hbm.at[pl.ds(base_b, CHUNK)], rows_v1)
        pltpu.sync_copy(cols_hbm.at[pl.ds(base_b, CHUNK)], cols_v1)
        pltpu.sync_copy(vals_hbm.at[pl.ds(base_b, CHUNK)], vals_v1)
        cp_b = pltpu.async_copy(x_hbm.at[cols_v1], xb1, sem1)
        cp_a.wait()
        _scale_and_scatter(xb0, vals_v0, rows_v0)
        cp_b.wait()
        _scale_and_scatter(xb1, vals_v1, rows_v1)
        return carry
    lax.fori_loop(0, NCHUNKS // 2, pair_body, 0)

    base_t = wid * EPT_PAD + (NCHUNKS - 1) * CHUNK
    pltpu.sync_copy(rows_hbm.at[pl.ds(base_t, CHUNK)], rows_v0)
    pltpu.sync_copy(cols_hbm.at[pl.ds(base_t, CHUNK)], cols_v0)
    pltpu.sync_copy(vals_hbm.at[pl.ds(base_t, CHUNK)], vals_v0)
    pltpu.async_copy(x_hbm.at[cols_v0], xb0, sem0).wait()
    _scale_and_scatter(xb0, vals_v0, rows_v0)

    plsc.subcore_barrier()

    @pl.when(jnp.logical_and(cid == 0, sid < NS - 1))
    def _():
        sl = pl.ds(sid * STRIPE, STRIPE)
        pltpu.sync_copy(agg_sh.at[sl], out0_hbm.at[sl])

    @pl.when(jnp.logical_and(cid == 0, sid == NS - 1))
    def _():
        sl = pl.ds(sid * STRIPE, LAST_STRIPE)
        pltpu.sync_copy(agg_sh.at[sl], out0_hbm.at[sl])

    @pl.when(jnp.logical_and(cid == 1, sid < NS - 1))
    def _():
        sl = pl.ds(sid * STRIPE, STRIPE)
        pltpu.sync_copy(agg_sh.at[sl], out1_hbm.at[sl])

    @pl.when(jnp.logical_and(cid == 1, sid == NS - 1))
    def _():
        sl = pl.ds(sid * STRIPE, LAST_STRIPE)
        pltpu.sync_copy(agg_sh.at[sl], out1_hbm.at[sl])


def _heads_body(p0_ref, p1_ref, wmu_ref, bmu_ref, wlv_ref, blv_ref,
                mu_ref, lv_ref):
    agg = p0_ref[...] + p1_ref[...]
    dn = (((1,), (1,)), ((), ()))
    mu = lax.dot_general(agg, wmu_ref[...], dn,
                         precision=lax.Precision.HIGHEST,
                         preferred_element_type=jnp.float32) + bmu_ref[...]
    lv = lax.dot_general(agg, wlv_ref[...], dn,
                         precision=lax.Precision.HIGHEST,
                         preferred_element_type=jnp.float32) + blv_ref[...]
    mu_ref[...] = mu
    lv_ref[...] = jnp.clip(lv, -10.0, 3.0)


_heads_call = pl.pallas_call(
    _heads_body,
    out_shape=[jax.ShapeDtypeStruct((N, D_LAT), jnp.float32),
               jax.ShapeDtypeStruct((N, D_LAT), jnp.float32)],
)


@functools.partial(
    pl.kernel,
    out_type=jax.ShapeDtypeStruct((E_PAD * 16 // 128, 128), jnp.float32),
    mesh=_mesh,
    scratch_types=[
        pltpu.VMEM((CHUNK,), jnp.int32),
        pltpu.VMEM((CHUNK,), jnp.int32),
        pltpu.VMEM((CHUNK, D_LAT), jnp.float32),
        pltpu.VMEM((CHUNK, D_LAT), jnp.float32),
        pltpu.VMEM((CHUNK,), jnp.int32),
        pltpu.VMEM((CHUNK,), jnp.int32),
        pltpu.VMEM((CHUNK, D_LAT), jnp.float32),
        pltpu.VMEM((CHUNK, D_LAT), jnp.float32),
        pltpu.VMEM((CHUNK * 16 // 128, 128), jnp.float32),
        pltpu.SemaphoreType.DMA,
        pltpu.SemaphoreType.DMA,
        pltpu.SemaphoreType.DMA,
        pltpu.SemaphoreType.DMA,
    ],
    compiler_params=pltpu.CompilerParams(use_tc_tiling_on_sc=False),
)
def _decoder_kernel(z_hbm, r_hbm, c_hbm, ph_hbm,
                    r_v0, c_v0, zr0, zc0, r_v1, c_v1, zr1, zc1,
                    sbuf, sem_r0, sem_c0, sem_r1, sem_c1):
    cid = lax.axis_index("c")
    sid = lax.axis_index("s")
    wid = cid * NS + sid

    def _dots_and_store(zr, zc, base):
        def edge_body(i, c2):
            s = jnp.zeros((16,), jnp.float32)
            for j in range(D_LAT // 16):
                s = s + zr[i, pl.ds(j * 16, 16)] * zc[i, pl.ds(j * 16, 16)]
            sbuf[i // 8, pl.ds((i % 8) * 16, 16)] = s
            return c2
        lax.fori_loop(0, CHUNK, edge_body, 0)
        pltpu.sync_copy(sbuf, ph_hbm.at[pl.ds(base // 8, CHUNK * 16 // 128)])

    def pair_body(p, carry):
        base_a = wid * EPT_PAD + (2 * p) * CHUNK
        base_b = base_a + CHUNK
        pltpu.sync_copy(r_hbm.at[pl.ds(base_a, CHUNK)], r_v0)
        pltpu.sync_copy(c_hbm.at[pl.ds(base_a, CHUNK)], c_v0)
        cp_r0 = pltpu.async_copy(z_hbm.at[r_v0], zr0, sem_r0)
        cp_c0 = pltpu.async_copy(z_hbm.at[c_v0], zc0, sem_c0)
        pltpu.sync_copy(r_hbm.at[pl.ds(base_b, CHUNK)], r_v1)
        pltpu.sync_copy(c_hbm.at[pl.ds(base_b, CHUNK)], c_v1)
        cp_r1 = pltpu.async_copy(z_hbm.at[r_v1], zr1, sem_r1)
        cp_c1 = pltpu.async_copy(z_hbm.at[c_v1], zc1, sem_c1)
        cp_r0.wait()
        cp_c0.wait()
        _dots_and_store(zr0, zc0, base_a)
        cp_r1.wait()
        cp_c1.wait()
        _dots_and_store(zr1, zc1, base_b)
        return carry
    lax.fori_loop(0, NCHUNKS // 2, pair_body, 0)

    base_t = wid * EPT_PAD + (NCHUNKS - 1) * CHUNK
    pltpu.sync_copy(r_hbm.at[pl.ds(base_t, CHUNK)], r_v0)
    pltpu.sync_copy(c_hbm.at[pl.ds(base_t, CHUNK)], c_v0)
    cp_r = pltpu.async_copy(z_hbm.at[r_v0], zr0, sem_r0)
    cp_c = pltpu.async_copy(z_hbm.at[c_v0], zc0, sem_c0)
    cp_r.wait()
    cp_c.wait()
    _dots_and_store(zr0, zc0, base_t)


_RBLK = 512
_RROWS = E_PAD * 16 // 128


def _reduce_body(ph_ref, out_ref):
    rows = lax.broadcasted_iota(jnp.int32, (128, 128), 0)
    cols = lax.broadcasted_iota(jnp.int32, (128, 128), 1)
    g = jnp.logical_and(rows // 16 == cols, cols < 8).astype(jnp.float32)
    out_ref[...] = lax.dot_general(ph_ref[...], g, (((1,), (0,)), ((), ())),
                                   precision=lax.Precision.HIGHEST,
                                   preferred_element_type=jnp.float32)


_reduce_call = pl.pallas_call(
    _reduce_body,
    grid=(_RROWS // _RBLK,),
    in_specs=[pl.BlockSpec((_RBLK, 128), lambda i: (i, 0))],
    out_specs=pl.BlockSpec((_RBLK, 128), lambda i: (i, 0)),
    out_shape=jax.ShapeDtypeStruct((_RROWS, 128), jnp.float32),
)


def kernel(x, adj_edge_index, adj_values, edge_index, W_mu, b_mu,
           W_logvar, b_logvar):
    pad = E_PAD - E
    ar = jnp.concatenate([adj_edge_index[0].astype(jnp.int32),
                          jnp.zeros((pad,), jnp.int32)])
    ac = jnp.concatenate([adj_edge_index[1].astype(jnp.int32),
                          jnp.zeros((pad,), jnp.int32)])
    av = jnp.concatenate([adj_values.astype(jnp.float32),
                          jnp.zeros((pad,), jnp.float32)])
    er = jnp.concatenate([edge_index[0].astype(jnp.int32),
                          jnp.zeros((pad,), jnp.int32)])
    ec = jnp.concatenate([edge_index[1].astype(jnp.int32),
                          jnp.zeros((pad,), jnp.int32)])
    zeros_blk = jnp.zeros((STRIPE, D_IN), jnp.float32)

    p0, p1 = _spmm_kernel(x, ar, ac, av, zeros_blk)
    mu, logvar = _heads_call(p0, p1, W_mu, b_mu.reshape(1, D_LAT),
                             W_logvar, b_logvar.reshape(1, D_LAT))
    ph = _decoder_kernel(mu, er, ec)
    dec_pad = _reduce_call(ph)[:, :8].reshape(E_PAD)
    return (dec_pad[:E], mu, logvar)
